# Initial kernel scaffold; baseline (speedup 1.0000x reference)
import jax
import jax.numpy as jnp
from jax.experimental import pallas as pl


def kernel(x, edge_index, edge, W1, b1, g1, be1, W2, b2):
    raise NotImplementedError("write your pallas kernel here")


# speedup vs baseline: 3.0279x; 3.0279x over previous
import functools

import jax
import jax.numpy as jnp
from jax import lax
from jax.experimental import pallas as pl
from jax.experimental.pallas import tpu as pltpu
from jax.experimental.pallas import tpu_sc as plsc

EPS = 1e-5
D = 128
CH = 128


def _proj_body(x_ref, wa_ref, wb_ref, a_ref, b_ref):
    x = x_ref[...]
    a_ref[...] = jnp.dot(x, wa_ref[...], preferred_element_type=jnp.float32)
    b_ref[...] = jnp.dot(x, wb_ref[...], preferred_element_type=jnp.float32)


def _node_projections(x, wa, wb):
    n = x.shape[0]
    bn = 2000 if n % 2000 == 0 else n
    grid = n // bn
    return pl.pallas_call(
        _proj_body,
        grid=(grid,),
        in_specs=[
            pl.BlockSpec((bn, D), lambda i: (i, 0)),
            pl.BlockSpec((D, D), lambda i: (0, 0)),
            pl.BlockSpec((D, D), lambda i: (0, 0)),
        ],
        out_specs=[
            pl.BlockSpec((bn, D), lambda i: (i, 0)),
            pl.BlockSpec((bn, D), lambda i: (i, 0)),
        ],
        out_shape=[
            jax.ShapeDtypeStruct((n, D), jnp.float32),
            jax.ShapeDtypeStruct((n, D), jnp.float32),
        ],
    )(x, wa, wb)


def _sc_gather(a, b, src, dst):
    e = src.shape[0]
    info = plsc.get_sparse_core_info()
    nc, ns = info.num_cores, info.num_subcores
    nw = nc * ns
    total_chunks = e // CH
    base_loops = total_chunks // nw
    extra = total_chunks % nw
    mesh = plsc.VectorSubcoreMesh(core_axis_name="c", subcore_axis_name="s")

    @functools.partial(
        pl.kernel,
        mesh=mesh,
        out_type=[
            jax.ShapeDtypeStruct((e, D), jnp.float32),
            jax.ShapeDtypeStruct((e, D), jnp.float32),
        ],
        scratch_types=[
            pltpu.VMEM((CH,), jnp.int32),
            pltpu.VMEM((CH,), jnp.int32),
            pltpu.VMEM((CH, D), jnp.float32),
            pltpu.VMEM((CH, D), jnp.float32),
            pltpu.SemaphoreType.DMA,
            pltpu.SemaphoreType.DMA,
        ],
    )
    def gather_kernel(a_hbm, b_hbm, src_hbm, dst_hbm, gs_hbm, gd_hbm,
                      idxs, idxd, bufa, bufb, sema, semb):
        wid = lax.axis_index("s") * nc + lax.axis_index("c")

        def body(i, carry):
            base = (wid + i * nw) * CH
            pltpu.sync_copy(src_hbm.at[pl.ds(base, CH)], idxs)
            pltpu.sync_copy(dst_hbm.at[pl.ds(base, CH)], idxd)
            ca = pltpu.async_copy(a_hbm.at[idxs], bufa, sema)
            cb = pltpu.async_copy(b_hbm.at[idxd], bufb, semb)
            ca.wait()
            cb.wait()
            pltpu.sync_copy(bufa, gs_hbm.at[pl.ds(base, CH)])
            pltpu.sync_copy(bufb, gd_hbm.at[pl.ds(base, CH)])
            return carry

        nloops = base_loops + jnp.where(wid < extra, 1, 0)
        lax.fori_loop(0, nloops, body, 0)

    return gather_kernel(a, b, src, dst)


def _mlp_body(gs_ref, gd_ref, e_ref, wc_ref, b1_ref, g1_ref, be1_ref,
              w2_ref, b2_ref, o_ref):
    eb = e_ref[...]
    h = (gs_ref[...] + gd_ref[...]
         + jnp.dot(eb, wc_ref[...], preferred_element_type=jnp.float32)
         + b1_ref[...])
    m = jnp.mean(h, axis=-1, keepdims=True)
    v = jnp.mean((h - m) ** 2, axis=-1, keepdims=True)
    hn = (h - m) / jnp.sqrt(v + EPS) * g1_ref[...] + be1_ref[...]
    hr = jnp.maximum(hn, 0.0)
    o_ref[...] = (eb + jnp.dot(hr, w2_ref[...], preferred_element_type=jnp.float32)
                  + b2_ref[...])


def _edge_mlp(gs, gd, edge, wc, b1, g1, be1, w2, b2):
    e = edge.shape[0]
    be = 2000 if e % 2000 == 0 else e
    grid = e // be
    row = lambda v: v.reshape(1, D)
    vec_spec = pl.BlockSpec((1, D), lambda i: (0, 0))
    mat_spec = pl.BlockSpec((D, D), lambda i: (0, 0))
    blk_spec = pl.BlockSpec((be, D), lambda i: (i, 0))
    return pl.pallas_call(
        _mlp_body,
        grid=(grid,),
        in_specs=[blk_spec, blk_spec, blk_spec, mat_spec,
                  vec_spec, vec_spec, vec_spec, mat_spec, vec_spec],
        out_specs=blk_spec,
        out_shape=jax.ShapeDtypeStruct((e, D), jnp.float32),
    )(gs, gd, edge, wc, row(b1), row(g1), row(be1), w2, row(b2))


def kernel(x, edge_index, edge, W1, b1, g1, be1, W2, b2):
    src = edge_index[0]
    dst = edge_index[1]
    wa, wb, wc = W1[:D], W1[D:2 * D], W1[2 * D:]
    a, b = _node_projections(x, wa, wb)
    gs, gd = _sc_gather(a, b, src, dst)
    return _edge_mlp(gs, gd, edge, wc, b1, g1, be1, W2, b2)


# speedup vs baseline: 3.4294x; 1.1326x over previous
import functools

import jax
import jax.numpy as jnp
from jax import lax
from jax.experimental import pallas as pl
from jax.experimental.pallas import tpu as pltpu
from jax.experimental.pallas import tpu_sc as plsc

EPS = 1e-5
D = 128
CH = 128


def _proj_body(x_ref, wa_ref, wb_ref, a_ref, b_ref):
    x = x_ref[...]
    a_ref[...] = jnp.dot(x, wa_ref[...], preferred_element_type=jnp.float32)
    b_ref[...] = jnp.dot(x, wb_ref[...], preferred_element_type=jnp.float32)


def _node_projections(x, wa, wb):
    n = x.shape[0]
    bn = 2000 if n % 2000 == 0 else n
    grid = n // bn
    return pl.pallas_call(
        _proj_body,
        grid=(grid,),
        in_specs=[
            pl.BlockSpec((bn, D), lambda i: (i, 0)),
            pl.BlockSpec((D, D), lambda i: (0, 0)),
            pl.BlockSpec((D, D), lambda i: (0, 0)),
        ],
        out_specs=[
            pl.BlockSpec((bn, D), lambda i: (i, 0)),
            pl.BlockSpec((bn, D), lambda i: (i, 0)),
        ],
        out_shape=[
            jax.ShapeDtypeStruct((n, D), jnp.float32),
            jax.ShapeDtypeStruct((n, D), jnp.float32),
        ],
    )(x, wa, wb)


def _sc_gather(a, b, src, dst):
    e = src.shape[0]
    info = plsc.get_sparse_core_info()
    nc, ns = info.num_cores, info.num_subcores
    nw = nc * ns
    total_chunks = e // CH
    iters = (total_chunks + nw - 1) // nw
    mesh = plsc.VectorSubcoreMesh(core_axis_name="c", subcore_axis_name="s")

    @functools.partial(
        pl.kernel,
        mesh=mesh,
        out_type=[
            jax.ShapeDtypeStruct((e, D), jnp.float32),
            jax.ShapeDtypeStruct((e, D), jnp.float32),
        ],
        scratch_types=[
            pltpu.VMEM((2, CH), jnp.int32),
            pltpu.VMEM((2, CH), jnp.int32),
            pltpu.VMEM((2, CH, D), jnp.float32),
            pltpu.VMEM((2, CH, D), jnp.float32),
        ] + [pltpu.SemaphoreType.DMA] * 8,
    )
    def gather_kernel(a_hbm, b_hbm, src_hbm, dst_hbm, gs_hbm, gd_hbm,
                      idxs, idxd, bufa, bufb,
                      sga0, sga1, sgb0, sgb1, ssa0, ssa1, ssb0, ssb1):
        sga = (sga0, sga1)
        sgb = (sgb0, sgb1)
        ssa = (ssa0, ssa1)
        ssb = (ssb0, ssb1)
        wid = lax.axis_index("s") * nc + lax.axis_index("c")

        def valid(i):
            return wid + i * nw < total_chunks

        def stage_and_gather(i, slot):
            base = (wid + i * nw) * CH
            pltpu.sync_copy(src_hbm.at[pl.ds(base, CH)], idxs.at[slot])
            pltpu.sync_copy(dst_hbm.at[pl.ds(base, CH)], idxd.at[slot])
            pltpu.async_copy(a_hbm.at[idxs.at[slot]], bufa.at[slot], sga[slot])
            pltpu.async_copy(b_hbm.at[idxd.at[slot]], bufb.at[slot], sgb[slot])

        def wait_gathers(slot):
            pltpu.make_async_copy(a_hbm.at[pl.ds(0, CH)], bufa.at[slot], sga[slot]).wait()
            pltpu.make_async_copy(b_hbm.at[pl.ds(0, CH)], bufb.at[slot], sgb[slot]).wait()

        def scatter(i, slot):
            base = (wid + i * nw) * CH
            pltpu.async_copy(bufa.at[slot], gs_hbm.at[pl.ds(base, CH)], ssa[slot])
            pltpu.async_copy(bufb.at[slot], gd_hbm.at[pl.ds(base, CH)], ssb[slot])

        def wait_scatters(slot):
            pltpu.make_async_copy(bufa.at[slot], gs_hbm.at[pl.ds(0, CH)], ssa[slot]).wait()
            pltpu.make_async_copy(bufb.at[slot], gd_hbm.at[pl.ds(0, CH)], ssb[slot]).wait()

        def step(i, slot, first):
            nslot = 1 - slot

            @pl.when(valid(i))
            def _():
                wait_gathers(slot)
                scatter(i, slot)

            @pl.when(valid(i + 1))
            def _():
                if not first:
                    wait_scatters(nslot)
                stage_and_gather(i + 1, nslot)

        stage_and_gather(0, 0)
        step(0, 0, first=True)

        def loop_body(j, carry):
            step(2 * j + 1, 1, first=False)
            step(2 * j + 2, 0, first=False)
            return carry

        lax.fori_loop(0, iters // 2, loop_body, 0)

        wait_scatters(0)
        wait_scatters(1)

    return gather_kernel(a, b, src, dst)


def _mlp_body(gs_ref, gd_ref, e_ref, wc_ref, b1_ref, g1_ref, be1_ref,
              w2_ref, b2_ref, o_ref):
    eb = e_ref[...]
    h = (gs_ref[...] + gd_ref[...]
         + jnp.dot(eb, wc_ref[...], preferred_element_type=jnp.float32)
         + b1_ref[...])
    m = jnp.mean(h, axis=-1, keepdims=True)
    v = jnp.mean((h - m) ** 2, axis=-1, keepdims=True)
    hn = (h - m) / jnp.sqrt(v + EPS) * g1_ref[...] + be1_ref[...]
    hr = jnp.maximum(hn, 0.0)
    o_ref[...] = (eb + jnp.dot(hr, w2_ref[...], preferred_element_type=jnp.float32)
                  + b2_ref[...])


def _edge_mlp(gs, gd, edge, wc, b1, g1, be1, w2, b2):
    e = edge.shape[0]
    be = 2000 if e % 2000 == 0 else e
    grid = e // be
    row = lambda v: v.reshape(1, D)
    vec_spec = pl.BlockSpec((1, D), lambda i: (0, 0))
    mat_spec = pl.BlockSpec((D, D), lambda i: (0, 0))
    blk_spec = pl.BlockSpec((be, D), lambda i: (i, 0))
    return pl.pallas_call(
        _mlp_body,
        grid=(grid,),
        in_specs=[blk_spec, blk_spec, blk_spec, mat_spec,
                  vec_spec, vec_spec, vec_spec, mat_spec, vec_spec],
        out_specs=blk_spec,
        out_shape=jax.ShapeDtypeStruct((e, D), jnp.float32),
    )(gs, gd, edge, wc, row(b1), row(g1), row(be1), w2, row(b2))


def kernel(x, edge_index, edge, W1, b1, g1, be1, W2, b2):
    src = edge_index[0]
    dst = edge_index[1]
    wa, wb, wc = W1[:D], W1[D:2 * D], W1[2 * D:]
    a, b = _node_projections(x, wa, wb)
    gs, gd = _sc_gather(a, b, src, dst)
    return _edge_mlp(gs, gd, edge, wc, b1, g1, be1, W2, b2)


# speedup vs baseline: 3.6730x; 1.0710x over previous
import functools

import jax
import jax.numpy as jnp
from jax import lax
from jax.experimental import pallas as pl
from jax.experimental.pallas import tpu as pltpu
from jax.experimental.pallas import tpu_sc as plsc

EPS = 1e-5
D = 128
CH = 128


def _proj_body(x_ref, wa_ref, wb_ref, a_ref, b_ref):
    x = x_ref[...]
    a_ref[...] = jnp.dot(x, wa_ref[...], preferred_element_type=jnp.float32)
    b_ref[...] = jnp.dot(x, wb_ref[...], preferred_element_type=jnp.float32)


def _node_projections(x, wa, wb):
    n = x.shape[0]
    bn = 2000 if n % 2000 == 0 else n
    grid = n // bn
    return pl.pallas_call(
        _proj_body,
        grid=(grid,),
        in_specs=[
            pl.BlockSpec((bn, D), lambda i: (i, 0)),
            pl.BlockSpec((D, D), lambda i: (0, 0)),
            pl.BlockSpec((D, D), lambda i: (0, 0)),
        ],
        out_specs=[
            pl.BlockSpec((bn, D), lambda i: (i, 0)),
            pl.BlockSpec((bn, D), lambda i: (i, 0)),
        ],
        out_shape=[
            jax.ShapeDtypeStruct((n, D), jnp.float32),
            jax.ShapeDtypeStruct((n, D), jnp.float32),
        ],
    )(x, wa, wb)


def _sc_gather(a, b, src, dst):
    e = src.shape[0]
    info = plsc.get_sparse_core_info()
    nc, ns = info.num_cores, info.num_subcores
    nw = nc * ns
    total_chunks = e // CH
    iters = (total_chunks + nw - 1) // nw
    mesh = plsc.VectorSubcoreMesh(core_axis_name="c", subcore_axis_name="s")

    @functools.partial(
        pl.kernel,
        mesh=mesh,
        out_type=jax.ShapeDtypeStruct((e, D), jnp.float32),
        scratch_types=[
            pltpu.VMEM((2, CH), jnp.int32),
            pltpu.VMEM((2, CH), jnp.int32),
            pltpu.VMEM((2, CH, D), jnp.float32),
            pltpu.VMEM((2, CH, D), jnp.float32),
        ] + [pltpu.SemaphoreType.DMA] * 6,
    )
    def gather_kernel(a_hbm, b_hbm, src_hbm, dst_hbm, g_hbm,
                      idxs, idxd, bufa, bufb,
                      sga0, sga1, sgb0, sgb1, ssa0, ssa1):
        sga = (sga0, sga1)
        sgb = (sgb0, sgb1)
        ssa = (ssa0, ssa1)
        wid = lax.axis_index("s") * nc + lax.axis_index("c")

        def valid(i):
            return wid + i * nw < total_chunks

        def stage_and_gather(i, slot):
            base = (wid + i * nw) * CH
            pltpu.sync_copy(src_hbm.at[pl.ds(base, CH)], idxs.at[slot])
            pltpu.sync_copy(dst_hbm.at[pl.ds(base, CH)], idxd.at[slot])
            pltpu.async_copy(a_hbm.at[idxs.at[slot]], bufa.at[slot], sga[slot])
            pltpu.async_copy(b_hbm.at[idxd.at[slot]], bufb.at[slot], sgb[slot])

        def wait_gathers(slot):
            pltpu.make_async_copy(a_hbm.at[pl.ds(0, CH)], bufa.at[slot], sga[slot]).wait()
            pltpu.make_async_copy(b_hbm.at[pl.ds(0, CH)], bufb.at[slot], sgb[slot]).wait()

        def add_rows(slot):
            def row(r, carry):
                for c in range(D // 16):
                    sl = pl.ds(c * 16, 16)
                    bufa[slot, r, sl] = bufa[slot, r, sl] + bufb[slot, r, sl]
                return carry

            lax.fori_loop(0, CH, row, 0)

        def scatter(i, slot):
            base = (wid + i * nw) * CH
            pltpu.async_copy(bufa.at[slot], g_hbm.at[pl.ds(base, CH)], ssa[slot])

        def wait_scatters(slot):
            pltpu.make_async_copy(bufa.at[slot], g_hbm.at[pl.ds(0, CH)], ssa[slot]).wait()

        def step(i, slot, first):
            nslot = 1 - slot

            @pl.when(valid(i))
            def _():
                wait_gathers(slot)

            @pl.when(valid(i + 1))
            def _():
                if not first:
                    wait_scatters(nslot)
                stage_and_gather(i + 1, nslot)

            @pl.when(valid(i))
            def _():
                add_rows(slot)
                scatter(i, slot)

        stage_and_gather(0, 0)
        step(0, 0, first=True)

        def loop_body(j, carry):
            step(2 * j + 1, 1, first=False)
            step(2 * j + 2, 0, first=False)
            return carry

        lax.fori_loop(0, iters // 2, loop_body, 0)

        wait_scatters(0)
        wait_scatters(1)

    return gather_kernel(a, b, src, dst)


def _mlp_body(g_ref, e_ref, wc_ref, b1_ref, g1_ref, be1_ref,
              w2_ref, b2_ref, o_ref):
    eb = e_ref[...]
    h = (g_ref[...]
         + jnp.dot(eb, wc_ref[...], preferred_element_type=jnp.float32)
         + b1_ref[...])
    m = jnp.mean(h, axis=-1, keepdims=True)
    v = jnp.mean((h - m) ** 2, axis=-1, keepdims=True)
    hn = (h - m) / jnp.sqrt(v + EPS) * g1_ref[...] + be1_ref[...]
    hr = jnp.maximum(hn, 0.0)
    o_ref[...] = (eb + jnp.dot(hr, w2_ref[...], preferred_element_type=jnp.float32)
                  + b2_ref[...])


def _edge_mlp(g, edge, wc, b1, g1, be1, w2, b2):
    e = edge.shape[0]
    be = 2000 if e % 2000 == 0 else e
    grid = e // be
    row = lambda v: v.reshape(1, D)
    vec_spec = pl.BlockSpec((1, D), lambda i: (0, 0))
    mat_spec = pl.BlockSpec((D, D), lambda i: (0, 0))
    blk_spec = pl.BlockSpec((be, D), lambda i: (i, 0))
    return pl.pallas_call(
        _mlp_body,
        grid=(grid,),
        in_specs=[blk_spec, blk_spec, mat_spec,
                  vec_spec, vec_spec, vec_spec, mat_spec, vec_spec],
        out_specs=blk_spec,
        out_shape=jax.ShapeDtypeStruct((e, D), jnp.float32),
    )(g, edge, wc, row(b1), row(g1), row(be1), w2, row(b2))


def kernel(x, edge_index, edge, W1, b1, g1, be1, W2, b2):
    src = edge_index[0]
    dst = edge_index[1]
    wa, wb, wc = W1[:D], W1[D:2 * D], W1[2 * D:]
    a, b = _node_projections(x, wa, wb)
    g = _sc_gather(a, b, src, dst)
    return _edge_mlp(g, edge, wc, b1, g1, be1, W2, b2)
